# baseline (speedup 1.0000x reference)
import functools

import jax
import jax.numpy as jnp
from jax import lax
from jax.experimental import pallas as pl
from jax.experimental.pallas import tpu as pltpu
from jax.experimental.pallas import tpu_sc as plsc

N = 10000
D = 128
E = 320000

NPAD = 10240
NC = 2
NS = 16
NW = NC * NS
K = 128
CH = 80
GROUPS = 10
PER_GROUP = CH // GROUPS
EPW = CH * K
EPAD = NW * EPW
ROWS_PER_SUB = NPAD // NS


def _sc_segment_sum(h_msg, srcr, dstr):
    mesh = plsc.VectorSubcoreMesh(core_axis_name="c", subcore_axis_name="s",
                                  num_cores=NC, num_subcores=NS)

    @functools.partial(
        pl.kernel,
        out_type=jax.ShapeDtypeStruct((NC, NPAD, D), jnp.float32),
        mesh=mesh,
        scratch_types=[
            pltpu.VMEM((CH, K), jnp.int32),
            pltpu.VMEM((2, PER_GROUP, K), jnp.int32),
            pltpu.VMEM((2, K, D), jnp.float32),
            pltpu.VMEM_SHARED((NPAD, D), jnp.float32),
            pltpu.SemaphoreType.DMA,
            pltpu.SemaphoreType.DMA,
            pltpu.SemaphoreType.DMA,
            pltpu.SemaphoreType.DMA,
        ],
    )
    def k(h_hbm, src_hbm, dst_hbm, out_hbm, src_v, dst_g, rows_v, acc,
          gsem0, gsem1, isem0, isem1):
        cid = lax.axis_index("c")
        sid = lax.axis_index("s")
        wid = sid * NC + cid

        z16 = jnp.zeros((16,), jnp.float32)

        def zrow(r, _):
            for c in range(D // 16):
                rows_v[0, r, pl.ds(c * 16, 16)] = z16
            return 0

        lax.fori_loop(0, K, zrow, 0)
        for t in range(ROWS_PER_SUB // K):
            pltpu.sync_copy(rows_v.at[0],
                            acc.at[pl.ds(sid * ROWS_PER_SUB + t * K, K)])
        pltpu.sync_copy(src_hbm.at[wid], src_v)
        plsc.subcore_barrier()

        gsems = (gsem0, gsem1)
        pltpu.async_copy(h_hbm.at[src_v.at[0]], rows_v.at[0], gsem0)
        pltpu.async_copy(dst_hbm.at[wid, pl.ds(0, PER_GROUP)], dst_g.at[0], isem0)

        def group(g, _):
            base = g * PER_GROUP
            even = (g & 1) == 0
            gsel = g & 1
            gn = jnp.minimum(g + 1, GROUPS - 1)

            @pl.when(even)
            def _():
                pltpu.make_async_copy(dst_hbm.at[wid, pl.ds(0, PER_GROUP)],
                                      dst_g.at[0], isem0).wait()
                pltpu.async_copy(dst_hbm.at[wid, pl.ds(gn * PER_GROUP, PER_GROUP)],
                                 dst_g.at[1], isem1)

            @pl.when(jnp.logical_not(even))
            def _():
                pltpu.make_async_copy(dst_hbm.at[wid, pl.ds(0, PER_GROUP)],
                                      dst_g.at[1], isem1).wait()
                pltpu.async_copy(dst_hbm.at[wid, pl.ds(gn * PER_GROUP, PER_GROUP)],
                                 dst_g.at[0], isem0)

            for i in range(PER_GROUP):
                j = base + i
                jn = jnp.minimum(j + 1, CH - 1)
                p = i % 2
                pn = (i + 1) % 2
                pltpu.async_copy(h_hbm.at[src_v.at[jn]], rows_v.at[pn], gsems[pn])
                pltpu.make_async_copy(h_hbm.at[src_v.at[0]], rows_v.at[p],
                                      gsems[p]).wait()
                pltpu.sync_copy(rows_v.at[p], acc.at[dst_g.at[gsel, i]], add=True)
            return 0

        lax.fori_loop(0, GROUPS, group, 0)
        pltpu.make_async_copy(h_hbm.at[src_v.at[0]], rows_v.at[0], gsem0).wait()
        pltpu.make_async_copy(dst_hbm.at[wid, pl.ds(0, PER_GROUP)],
                              dst_g.at[0], isem0).wait()
        plsc.subcore_barrier()
        pltpu.sync_copy(acc.at[pl.ds(sid * ROWS_PER_SUB, ROWS_PER_SUB)],
                        out_hbm.at[cid, pl.ds(sid * ROWS_PER_SUB, ROWS_PER_SUB)])

    return k(h_msg, srcr, dstr)


def _bf16_rtne(a):
    u = lax.bitcast_convert_type(a, jnp.uint32)
    u = (u + jnp.uint32(0x7FFF) + ((u >> 16) & jnp.uint32(1))) & jnp.uint32(0xFFFF0000)
    return lax.bitcast_convert_type(u, jnp.float32)


def _gru_body_msg(parts_ref, h_ref, wm_ref, wiT_ref, whT_ref, bi_ref, bh_ref,
                  out_ref, msg_ref):
    _gru_common(parts_ref, h_ref, wm_ref, wiT_ref, whT_ref, bi_ref, bh_ref,
                out_ref, msg_ref)


def _gru_body_last(parts_ref, h_ref, wm_ref, wiT_ref, whT_ref, bi_ref, bh_ref,
                   out_ref):
    _gru_common(parts_ref, h_ref, wm_ref, wiT_ref, whT_ref, bi_ref, bh_ref,
                out_ref, None)


def _gru_common(parts_ref, h_ref, wm_ref, wiT_ref, whT_ref, bi_ref, bh_ref,
                out_ref, msg_ref):
    s = parts_ref[0] + parts_ref[1]
    agg = jnp.dot(s, wm_ref[...], preferred_element_type=jnp.float32,
                  precision=jax.lax.Precision.HIGHEST)
    gi = jnp.dot(agg, wiT_ref[...], preferred_element_type=jnp.float32) + bi_ref[...]
    h = h_ref[...]
    gh = jnp.dot(h, whT_ref[...], preferred_element_type=jnp.float32) + bh_ref[...]
    r = jax.nn.sigmoid(gi[:, :D] + gh[:, :D])
    z = jax.nn.sigmoid(gi[:, D:2 * D] + gh[:, D:2 * D])
    n = jnp.tanh(gi[:, 2 * D:] + r * gh[:, 2 * D:])
    out = jnp.maximum((1.0 - z) * n + z * h, 0.0)
    out_ref[...] = out
    if msg_ref is not None:
        msg_ref[...] = _bf16_rtne(out)


def _tc_gru(parts, h_pad, Wm_r, WiT, WhT, bi, bh, want_msg):
    B = 1024
    grid = (NPAD // B,)
    in_specs = [
        pl.BlockSpec((NC, B, D), lambda i: (0, i, 0)),
        pl.BlockSpec((B, D), lambda i: (i, 0)),
        pl.BlockSpec((D, D), lambda i: (0, 0)),
        pl.BlockSpec((D, 3 * D), lambda i: (0, 0)),
        pl.BlockSpec((D, 3 * D), lambda i: (0, 0)),
        pl.BlockSpec((1, 3 * D), lambda i: (0, 0)),
        pl.BlockSpec((1, 3 * D), lambda i: (0, 0)),
    ]
    if want_msg:
        return pl.pallas_call(
            _gru_body_msg,
            grid=grid,
            in_specs=in_specs,
            out_specs=[pl.BlockSpec((B, D), lambda i: (i, 0)),
                       pl.BlockSpec((B, D), lambda i: (i, 0))],
            out_shape=[jax.ShapeDtypeStruct((NPAD, D), jnp.float32),
                       jax.ShapeDtypeStruct((NPAD, D), jnp.float32)],
        )(parts, h_pad, Wm_r, WiT, WhT, bi, bh)
    return pl.pallas_call(
        _gru_body_last,
        grid=grid,
        in_specs=in_specs,
        out_specs=pl.BlockSpec((B, D), lambda i: (i, 0)),
        out_shape=jax.ShapeDtypeStruct((NPAD, D), jnp.float32),
    )(parts, h_pad, Wm_r, WiT, WhT, bi, bh), None


def kernel(x, edge_index, Wm1, Wi1, Wh1, bi1, bh1, Wm2, Wi2, Wh2, bi2, bh2,
           Wm3, Wi3, Wh3, bi3, bh3):
    src = edge_index[0].astype(jnp.int32)
    dst = edge_index[1].astype(jnp.int32)
    npad_extra = NPAD - N
    pad_len = EPAD - E
    pad_idx = N + jnp.arange(pad_len, dtype=jnp.int32) % npad_extra
    srcr = jnp.concatenate([src, pad_idx]).reshape(NW, CH, K)
    dstr = jnp.concatenate([dst, pad_idx]).reshape(NW, CH, K)

    h_pad = jnp.pad(x, ((0, npad_extra), (0, 0)))
    h_msg = _bf16_rtne(h_pad)

    layers = ((Wm1, Wi1, Wh1, bi1, bh1),
              (Wm2, Wi2, Wh2, bi2, bh2),
              (Wm3, Wi3, Wh3, bi3, bh3))
    for li, (Wm, Wi, Wh, bi, bh) in enumerate(layers):
        parts = _sc_segment_sum(h_msg, srcr, dstr)
        h_pad, h_msg = _tc_gru(parts, h_pad, _bf16_rtne(Wm), Wi.T, Wh.T,
                               bi.reshape(1, 3 * D), bh.reshape(1, 3 * D),
                               want_msg=(li < 2))
    return h_pad[:N]


# speedup vs baseline: 1.0095x; 1.0095x over previous
import functools

import jax
import jax.numpy as jnp
from jax import lax
from jax.experimental import pallas as pl
from jax.experimental.pallas import tpu as pltpu
from jax.experimental.pallas import tpu_sc as plsc

N = 10000
D = 128
E = 320000

NPAD = 10240
NC = 2
NS = 16
NW = NC * NS
K = 128
CH = 80
GROUPS = 10
PER_GROUP = CH // GROUPS
EPW = CH * K
EPAD = NW * EPW
ROWS_PER_SUB = NPAD // NS


def _sc_segment_sum(h_msg, srcr, dstr):
    mesh = plsc.VectorSubcoreMesh(core_axis_name="c", subcore_axis_name="s",
                                  num_cores=NC, num_subcores=NS)

    @functools.partial(
        pl.kernel,
        out_type=jax.ShapeDtypeStruct((NC, NPAD, D), jnp.float32),
        mesh=mesh,
        scratch_types=[
            pltpu.VMEM((CH, K), jnp.int32),
            pltpu.VMEM((2, PER_GROUP, K), jnp.int32),
            pltpu.VMEM((2, K, D), jnp.float32),
            pltpu.VMEM_SHARED((NPAD, D), jnp.float32),
            pltpu.SemaphoreType.DMA,
            pltpu.SemaphoreType.DMA,
            pltpu.SemaphoreType.DMA,
            pltpu.SemaphoreType.DMA,
        ],
    )
    def k(h_hbm, src_hbm, dst_hbm, out_hbm, src_v, dst_g, rows_v, acc,
          gsem0, gsem1, isem0, isem1):
        cid = lax.axis_index("c")
        sid = lax.axis_index("s")
        wid = sid * NC + cid

        z16 = jnp.zeros((16,), jnp.float32)

        def zrow(r, _):
            for c in range(D // 16):
                rows_v[0, r, pl.ds(c * 16, 16)] = z16
            return 0

        lax.fori_loop(0, K, zrow, 0)
        for t in range(ROWS_PER_SUB // K):
            pltpu.sync_copy(rows_v.at[0],
                            acc.at[pl.ds(sid * ROWS_PER_SUB + t * K, K)])
        pltpu.sync_copy(src_hbm.at[wid], src_v)
        plsc.subcore_barrier()

        gsems = (gsem0, gsem1)
        pltpu.async_copy(h_hbm.at[src_v.at[0]], rows_v.at[0], gsem0)
        pltpu.async_copy(dst_hbm.at[wid, pl.ds(0, PER_GROUP)], dst_g.at[0], isem0)

        def group(g, _):
            base = g * PER_GROUP
            even = (g & 1) == 0
            gsel = g & 1
            gn = jnp.minimum(g + 1, GROUPS - 1)

            @pl.when(even)
            def _():
                pltpu.make_async_copy(dst_hbm.at[wid, pl.ds(0, PER_GROUP)],
                                      dst_g.at[0], isem0).wait()
                pltpu.async_copy(dst_hbm.at[wid, pl.ds(gn * PER_GROUP, PER_GROUP)],
                                 dst_g.at[1], isem1)

            @pl.when(jnp.logical_not(even))
            def _():
                pltpu.make_async_copy(dst_hbm.at[wid, pl.ds(0, PER_GROUP)],
                                      dst_g.at[1], isem1).wait()
                pltpu.async_copy(dst_hbm.at[wid, pl.ds(gn * PER_GROUP, PER_GROUP)],
                                 dst_g.at[0], isem0)

            for i in range(PER_GROUP):
                j = base + i
                jn = jnp.minimum(j + 1, CH - 1)
                p = i % 2
                pn = (i + 1) % 2
                pltpu.async_copy(h_hbm.at[src_v.at[jn]], rows_v.at[pn], gsems[pn])
                pltpu.make_async_copy(h_hbm.at[src_v.at[0]], rows_v.at[p],
                                      gsems[p]).wait()
                pltpu.sync_copy(rows_v.at[p], acc.at[dst_g.at[gsel, i]], add=True)
            return 0

        lax.fori_loop(0, GROUPS, group, 0)
        pltpu.make_async_copy(h_hbm.at[src_v.at[0]], rows_v.at[0], gsem0).wait()
        pltpu.make_async_copy(dst_hbm.at[wid, pl.ds(0, PER_GROUP)],
                              dst_g.at[0], isem0).wait()
        plsc.subcore_barrier()
        pltpu.sync_copy(acc.at[pl.ds(sid * ROWS_PER_SUB, ROWS_PER_SUB)],
                        out_hbm.at[cid, pl.ds(sid * ROWS_PER_SUB, ROWS_PER_SUB)])

    return k(h_msg, srcr, dstr)


def _bf16_rtne(a):
    u = lax.bitcast_convert_type(a, jnp.uint32)
    u = (u + jnp.uint32(0x7FFF) + ((u >> 16) & jnp.uint32(1))) & jnp.uint32(0xFFFF0000)
    return lax.bitcast_convert_type(u, jnp.float32)


def _gru_body_msg(parts_ref, h_ref, wm_ref, wiT_ref, whT_ref, bi_ref, bh_ref,
                  out_ref, msg_ref):
    _gru_common(parts_ref, h_ref, wm_ref, wiT_ref, whT_ref, bi_ref, bh_ref,
                out_ref, msg_ref)


def _gru_body_last(parts_ref, h_ref, wm_ref, wiT_ref, whT_ref, bi_ref, bh_ref,
                   out_ref):
    _gru_common(parts_ref, h_ref, wm_ref, wiT_ref, whT_ref, bi_ref, bh_ref,
                out_ref, None)


def _gru_common(parts_ref, h_ref, wm_ref, wiT_ref, whT_ref, bi_ref, bh_ref,
                out_ref, msg_ref):
    s = parts_ref[0] + parts_ref[1]
    agg = jnp.dot(s, wm_ref[...], preferred_element_type=jnp.float32,
                  precision=jax.lax.Precision.HIGHEST)
    gi = jnp.dot(agg, wiT_ref[...], preferred_element_type=jnp.float32) + bi_ref[...]
    h = h_ref[...]
    gh = jnp.dot(h, whT_ref[...], preferred_element_type=jnp.float32) + bh_ref[...]
    r = jax.nn.sigmoid(gi[:, :D] + gh[:, :D])
    z = jax.nn.sigmoid(gi[:, D:2 * D] + gh[:, D:2 * D])
    n = jnp.tanh(gi[:, 2 * D:] + r * gh[:, 2 * D:])
    out = jnp.maximum((1.0 - z) * n + z * h, 0.0)
    out_ref[...] = out
    if msg_ref is not None:
        msg_ref[...] = _bf16_rtne(out)


def _tc_gru(parts, h_pad, Wm_r, WiT, WhT, bi, bh, want_msg):
    B = 2048
    grid = (NPAD // B,)
    in_specs = [
        pl.BlockSpec((NC, B, D), lambda i: (0, i, 0)),
        pl.BlockSpec((B, D), lambda i: (i, 0)),
        pl.BlockSpec((D, D), lambda i: (0, 0)),
        pl.BlockSpec((D, 3 * D), lambda i: (0, 0)),
        pl.BlockSpec((D, 3 * D), lambda i: (0, 0)),
        pl.BlockSpec((1, 3 * D), lambda i: (0, 0)),
        pl.BlockSpec((1, 3 * D), lambda i: (0, 0)),
    ]
    if want_msg:
        return pl.pallas_call(
            _gru_body_msg,
            grid=grid,
            in_specs=in_specs,
            out_specs=[pl.BlockSpec((B, D), lambda i: (i, 0)),
                       pl.BlockSpec((B, D), lambda i: (i, 0))],
            out_shape=[jax.ShapeDtypeStruct((NPAD, D), jnp.float32),
                       jax.ShapeDtypeStruct((NPAD, D), jnp.float32)],
        )(parts, h_pad, Wm_r, WiT, WhT, bi, bh)
    return pl.pallas_call(
        _gru_body_last,
        grid=grid,
        in_specs=in_specs,
        out_specs=pl.BlockSpec((B, D), lambda i: (i, 0)),
        out_shape=jax.ShapeDtypeStruct((NPAD, D), jnp.float32),
    )(parts, h_pad, Wm_r, WiT, WhT, bi, bh), None


def kernel(x, edge_index, Wm1, Wi1, Wh1, bi1, bh1, Wm2, Wi2, Wh2, bi2, bh2,
           Wm3, Wi3, Wh3, bi3, bh3):
    src = edge_index[0].astype(jnp.int32)
    dst = edge_index[1].astype(jnp.int32)
    npad_extra = NPAD - N
    pad_len = EPAD - E
    pad_idx = N + jnp.arange(pad_len, dtype=jnp.int32) % npad_extra
    srcr = jnp.concatenate([src, pad_idx]).reshape(NW, CH, K)
    dstr = jnp.concatenate([dst, pad_idx]).reshape(NW, CH, K)

    h_pad = jnp.pad(x, ((0, npad_extra), (0, 0)))
    h_msg = _bf16_rtne(h_pad)

    layers = ((Wm1, Wi1, Wh1, bi1, bh1),
              (Wm2, Wi2, Wh2, bi2, bh2),
              (Wm3, Wi3, Wh3, bi3, bh3))
    for li, (Wm, Wi, Wh, bi, bh) in enumerate(layers):
        parts = _sc_segment_sum(h_msg, srcr, dstr)
        h_pad, h_msg = _tc_gru(parts, h_pad, _bf16_rtne(Wm), Wi.T, Wh.T,
                               bi.reshape(1, 3 * D), bh.reshape(1, 3 * D),
                               want_msg=(li < 2))
    return h_pad[:N]
